# baseline (speedup 1.0000x reference)
import functools

import jax
import jax.numpy as jnp
from jax import lax
from jax.experimental import pallas as pl
from jax.experimental.pallas import tpu as pltpu
from jax.experimental.pallas import tpu_sc as plsc

B = 16
C = 64
K = 512
D = 256
P = B * C
L = 16

C_SC = 8
NWSC = C_SC * 2
DH = D // 2
DCH = 32
NCH = DH // DCH
NBUF = 3

C_TC = C - C_SC
GS = 8
NG = C_TC // GS


@functools.partial(
    pl.kernel,
    out_type=jax.ShapeDtypeStruct((C_SC, 2, B, DH), jnp.float32),
    mesh=plsc.VectorSubcoreMesh(core_axis_name="c", subcore_axis_name="s"),
    scratch_types=[
        pltpu.VMEM((B, C_SC, K), jnp.float32),
        pltpu.VMEM((DCH, K), jnp.float32),
        pltpu.VMEM((DCH, K), jnp.float32),
        pltpu.VMEM((DCH, K), jnp.float32),
        pltpu.VMEM((B, DH), jnp.float32),
        pltpu.VMEM((B * L,), jnp.int32),
        pltpu.SemaphoreType.DMA,
        pltpu.SemaphoreType.DMA,
        pltpu.SemaphoreType.DMA,
        pltpu.SemaphoreType.DMA,
    ],
    compiler_params=pltpu.CompilerParams(needs_layout_passes=False),
)
def _vq_sc_kernel(cat_hbm, emb_hbm, out_hbm, cat_v, strip0, strip1, strip2,
                  res_v, ctab_v, sem0, sem1, sem2, semc):
    wid = lax.axis_index("s") * 2 + lax.axis_index("c")

    @pl.when(wid < NWSC)
    def _():
        s = wid // 2
        h = wid % 2
        col0 = pl.multiple_of(s * K, K)
        row0 = pl.multiple_of(h * DH, DH)

        strips = (strip0, strip1, strip2)
        sems = (sem0, sem1, sem2)

        copies = [None] * NBUF
        for t in range(NBUF):
            copies[t] = pltpu.async_copy(
                emb_hbm.at[pl.ds(row0 + t * DCH, DCH), pl.ds(col0, K)],
                strips[t], sems[t],
            )
        cat_cp = pltpu.async_copy(cat_hbm.at[:, pl.ds(0, C_SC), :], cat_v, semc)

        lanes = lax.iota(jnp.int32, L)
        svec = jnp.full((L,), s, jnp.int32)

        cat_cp.wait()

        def cls_body(j, carry):
            vmax, vidx = carry
            for u in range(8):
                jj = j * 8 + u
                v = plsc.load_gather(
                    cat_v, [lanes, svec, jnp.full((L,), jj, jnp.int32)]
                )
                gt = v > vmax
                vmax = jnp.where(gt, v, vmax)
                vidx = jnp.where(gt, jj, vidx)
            return vmax, vidx

        vmax0 = jnp.full((L,), -jnp.inf, jnp.float32)
        vidx0 = jnp.zeros((L,), jnp.int32)
        _, vidx = lax.fori_loop(0, K // 8, cls_body, (vmax0, vidx0))

        for dl in range(L):
            plsc.store_scatter(ctab_v, [lanes * L + dl], vidx)

        for t in range(NCH):
            copies[t % NBUF].wait()
            strip = strips[t % NBUF]

            def pair_body(p, carry, _t=t, _strip=strip):
                cvec = ctab_v[pl.ds(p * L, L)]
                v0 = plsc.load_gather(_strip, [lanes, cvec])
                v1 = plsc.load_gather(_strip, [lanes + L, cvec])
                res_v[p, pl.ds(_t * DCH, L)] = v0
                res_v[p, pl.ds(_t * DCH + L, L)] = v1
                return carry

            lax.fori_loop(0, B, pair_body, 0)

            if t + NBUF < NCH:
                copies[t % NBUF] = pltpu.async_copy(
                    emb_hbm.at[pl.ds(row0 + (t + NBUF) * DCH, DCH),
                               pl.ds(col0, K)],
                    strips[t % NBUF], sems[t % NBUF],
                )

        pltpu.sync_copy(res_v, out_hbm.at[s, h])


def _vq_tc_body(cat_ref, emb_ref, out_ref):
    cat = cat_ref[...]
    m = jnp.max(cat, axis=2, keepdims=True)
    kio = lax.broadcasted_iota(jnp.int32, (B, GS, K), 2)
    idx = jnp.min(jnp.where(cat == m, kio, K), axis=2)
    flat = idx + lax.broadcasted_iota(jnp.int32, (B, GS), 1) * K
    oh = (flat.reshape(B * GS, 1)
          == lax.broadcasted_iota(jnp.int32, (B * GS, GS * K), 1))
    q = jax.lax.dot_general(
        oh.astype(jnp.float32), emb_ref[...],
        (((1,), (1,)), ((), ())), preferred_element_type=jnp.float32,
    )
    out_ref[...] = q.reshape(B, GS, D)


_vq_tc_kernel = pl.pallas_call(
    _vq_tc_body,
    grid=(NG,),
    in_specs=[
        pl.BlockSpec((B, GS, K), lambda i: (0, i + C_SC // GS, 0)),
        pl.BlockSpec((D, GS * K), lambda i: (0, i + C_SC // GS)),
    ],
    out_specs=pl.BlockSpec((B, GS, D), lambda i: (0, i, 0)),
    out_shape=jax.ShapeDtypeStruct((B, C_TC, D), jnp.float32),
)


def kernel(categoricals_onehot, embeddings):
    sc = _vq_sc_kernel(categoricals_onehot, embeddings)
    tc = _vq_tc_kernel(categoricals_onehot, embeddings)
    sc_bm = sc.transpose(2, 0, 1, 3).reshape(B, C_SC, D)
    return jnp.concatenate([sc_bm, tc], axis=1).reshape(B, 8, 8, D)


# speedup vs baseline: 1.0061x; 1.0061x over previous
import functools

import jax
import jax.numpy as jnp
from jax import lax
from jax.experimental import pallas as pl
from jax.experimental.pallas import tpu as pltpu
from jax.experimental.pallas import tpu_sc as plsc

B = 16
C = 64
K = 512
D = 256
P = B * C
L = 16

C_SC = 8
NWSC = C_SC * 2
DH = D // 2
DCH = 32
NCH = DH // DCH
NBUF = 3

C_TC = C - C_SC
GS = 8
NG = C_TC // GS


@functools.partial(
    pl.kernel,
    out_type=jax.ShapeDtypeStruct((C_SC, B, D), jnp.float32),
    mesh=plsc.VectorSubcoreMesh(core_axis_name="c", subcore_axis_name="s"),
    scratch_types=[
        pltpu.VMEM((B, C_SC, K), jnp.float32),
        pltpu.VMEM((DCH, K), jnp.float32),
        pltpu.VMEM((DCH, K), jnp.float32),
        pltpu.VMEM((DCH, K), jnp.float32),
        pltpu.VMEM((B, DH), jnp.float32),
        pltpu.VMEM((B * L,), jnp.int32),
        pltpu.SemaphoreType.DMA,
        pltpu.SemaphoreType.DMA,
        pltpu.SemaphoreType.DMA,
        pltpu.SemaphoreType.DMA,
    ],
    compiler_params=pltpu.CompilerParams(needs_layout_passes=False),
)
def _vq_sc_kernel(cat_hbm, emb_hbm, out_hbm, cat_v, strip0, strip1, strip2,
                  res_v, ctab_v, sem0, sem1, sem2, semc):
    wid = lax.axis_index("s") * 2 + lax.axis_index("c")

    @pl.when(wid < NWSC)
    def _():
        s = wid // 2
        h = wid % 2
        col0 = pl.multiple_of(s * K, K)
        row0 = pl.multiple_of(h * DH, DH)

        strips = (strip0, strip1, strip2)
        sems = (sem0, sem1, sem2)

        copies = [None] * NBUF
        for t in range(NBUF):
            copies[t] = pltpu.async_copy(
                emb_hbm.at[pl.ds(row0 + t * DCH, DCH), pl.ds(col0, K)],
                strips[t], sems[t],
            )
        cat_cp = pltpu.async_copy(cat_hbm.at[:, pl.ds(0, C_SC), :], cat_v, semc)

        lanes = lax.iota(jnp.int32, L)
        svec = jnp.full((L,), s, jnp.int32)

        with jax.named_scope("argmax"):
            cat_cp.wait()

            def cls_body(j, carry):
                vmax, vidx = carry
                for u in range(8):
                    jj = j * 8 + u
                    v = plsc.load_gather(
                        cat_v, [lanes, svec, jnp.full((L,), jj, jnp.int32)]
                    )
                    gt = v > vmax
                    vmax = jnp.where(gt, v, vmax)
                    vidx = jnp.where(gt, jj, vidx)
                return vmax, vidx

            vmax0 = jnp.full((L,), -jnp.inf, jnp.float32)
            vidx0 = jnp.zeros((L,), jnp.int32)
            _, vidx = lax.fori_loop(0, K // 8, cls_body, (vmax0, vidx0))

        with jax.named_scope("ctab"):
            for dl in range(L):
                plsc.store_scatter(ctab_v, [lanes * L + dl], vidx)

        with jax.named_scope("extract"):
            for t in range(NCH):
                copies[t % NBUF].wait()
                strip = strips[t % NBUF]

                def pair_body(p, carry, _t=t, _strip=strip):
                    cvec = ctab_v[pl.ds(p * L, L)]
                    v0 = plsc.load_gather(_strip, [lanes, cvec])
                    v1 = plsc.load_gather(_strip, [lanes + L, cvec])
                    res_v[p, pl.ds(_t * DCH, L)] = v0
                    res_v[p, pl.ds(_t * DCH + L, L)] = v1
                    return carry

                lax.fori_loop(0, B, pair_body, 0)

                if t + NBUF < NCH:
                    copies[t % NBUF] = pltpu.async_copy(
                        emb_hbm.at[pl.ds(row0 + (t + NBUF) * DCH, DCH),
                                   pl.ds(col0, K)],
                        strips[t % NBUF], sems[t % NBUF],
                    )

        with jax.named_scope("writeback"):
            pltpu.sync_copy(res_v, out_hbm.at[s, :, pl.ds(row0, DH)])


def _vq_tc_body(cat_ref, emb_ref, out_ref):
    cat = cat_ref[...]
    m = jnp.max(cat, axis=2, keepdims=True)
    kio = lax.broadcasted_iota(jnp.int32, (B, GS, K), 2)
    idx = jnp.min(jnp.where(cat == m, kio, K), axis=2)
    flat = idx + lax.broadcasted_iota(jnp.int32, (B, GS), 1) * K
    oh = (flat.reshape(B * GS, 1)
          == lax.broadcasted_iota(jnp.int32, (B * GS, GS * K), 1))
    q = jax.lax.dot_general(
        oh.astype(jnp.float32), emb_ref[...],
        (((1,), (1,)), ((), ())), preferred_element_type=jnp.float32,
    )
    out_ref[...] = q.reshape(B, GS, D)


_vq_tc_kernel = pl.pallas_call(
    _vq_tc_body,
    grid=(NG,),
    in_specs=[
        pl.BlockSpec((B, GS, K), lambda i: (0, i + C_SC // GS, 0)),
        pl.BlockSpec((D, GS * K), lambda i: (0, i + C_SC // GS)),
    ],
    out_specs=pl.BlockSpec((B, GS, D), lambda i: (0, i, 0)),
    out_shape=jax.ShapeDtypeStruct((B, C_TC, D), jnp.float32),
)


def kernel(categoricals_onehot, embeddings):
    sc = _vq_sc_kernel(categoricals_onehot, embeddings)
    tc = _vq_tc_kernel(categoricals_onehot, embeddings)
    sc_bm = jnp.swapaxes(sc, 0, 1)
    return jnp.concatenate([sc_bm, tc], axis=1).reshape(B, 8, 8, D)
